# Initial kernel scaffold; baseline (speedup 1.0000x reference)
import jax
import jax.numpy as jnp
from jax.experimental import pallas as pl


def kernel(features, edge_index, relative_node_idx, W1, b1, W2, b2, W3, b3):
    raise NotImplementedError("write your pallas kernel here")


# speedup vs baseline: 6.2052x; 6.2052x over previous
import functools

import jax
import jax.numpy as jnp
from jax import lax
from jax.experimental import pallas as pl
from jax.experimental.pallas import tpu as pltpu
from jax.experimental.pallas import tpu_sc as plsc

N_NODES = 10000
N_EDGES = 320000
D_IN = 128
H1 = 32
H2 = 16
N_SUB = 1024

NC = 2
NS = 16
NW = NC * NS
L = 16

NPAD = 10240
EPAD = NW * 10240
EPT = EPAD // NW
ECHUNK = 128
NCHUNK = EPT // ECHUNK
RPT = NPAD // NS
ZPT = N_SUB // NW


DW = 16


def _make_sc_pass(with_deg):
    mesh = plsc.VectorSubcoreMesh(core_axis_name="c", subcore_axis_name="s")
    out_type = [jax.ShapeDtypeStruct((NC, NPAD, H1), jnp.float32)]
    scratch = [
        pltpu.VMEM_SHARED((NPAD, H1), jnp.float32),
        pltpu.VMEM((ECHUNK,), jnp.int32),
        pltpu.VMEM((ECHUNK,), jnp.int32),
        pltpu.VMEM((ECHUNK, H1), jnp.float32),
        pltpu.VMEM((RPT, H1), jnp.float32),
        pltpu.SemaphoreType.DMA,
    ]
    if with_deg:
        out_type = out_type + [jax.ShapeDtypeStruct((NC, NPAD, DW), jnp.float32)]
        scratch = scratch + [
            pltpu.VMEM_SHARED((NPAD, DW), jnp.float32),
            pltpu.VMEM((ECHUNK, DW), jnp.float32),
            pltpu.VMEM((RPT, DW), jnp.float32),
        ]

    @functools.partial(
        pl.kernel, out_type=tuple(out_type), mesh=mesh, scratch_types=scratch,
        compiler_params=pltpu.CompilerParams(use_tc_tiling_on_sc=False))
    def sc_pass(table_hbm, src_hbm, dst_hbm, out_hbm, *rest):
        if with_deg:
            (degp_hbm, acc_sh, src_v, dst_v, rows_v, stage_v, sem,
             deg_sh, ones_v, dstage_v) = rest
        else:
            acc_sh, src_v, dst_v, rows_v, stage_v, sem = rest
        cid = lax.axis_index("c")
        sid = lax.axis_index("s")
        rbase = sid * RPT
        pltpu.sync_copy(table_hbm.at[pl.ds(rbase, RPT)], stage_v)
        pltpu.sync_copy(stage_v, acc_sh.at[pl.ds(rbase, RPT)])

        if with_deg:
            ones16 = jnp.full((L,), 1.0, jnp.float32)
            zeros16 = jnp.zeros((L,), jnp.float32)

            def floop(i, c):
                ones_v[i, :] = ones16
                return c

            lax.fori_loop(0, ECHUNK, floop, 0)

            def zloop(i, c):
                dstage_v[i, :] = zeros16
                return c

            lax.fori_loop(0, RPT, zloop, 0)
            pltpu.sync_copy(dstage_v, deg_sh.at[pl.ds(rbase, RPT)])
        plsc.subcore_barrier()

        ebase = (cid * NS + sid) * EPT

        def chunk(i, carry):
            e0 = ebase + i * ECHUNK
            pltpu.sync_copy(src_hbm.at[pl.ds(e0, ECHUNK)], src_v)
            pltpu.sync_copy(dst_hbm.at[pl.ds(e0, ECHUNK)], dst_v)
            pltpu.async_copy(table_hbm.at[src_v], rows_v, sem).wait()
            pltpu.sync_copy(rows_v, acc_sh.at[dst_v], add=True)
            if with_deg:
                pltpu.sync_copy(ones_v, deg_sh.at[dst_v], add=True)
            return carry

        lax.fori_loop(0, NCHUNK, chunk, 0)
        plsc.subcore_barrier()
        pltpu.sync_copy(acc_sh.at[pl.ds(rbase, RPT)], stage_v)
        pltpu.sync_copy(stage_v, out_hbm.at[cid].at[pl.ds(rbase, RPT)])
        if with_deg:
            pltpu.sync_copy(deg_sh.at[pl.ds(rbase, RPT)], dstage_v)
            pltpu.sync_copy(dstage_v, degp_hbm.at[cid].at[pl.ds(rbase, RPT)])

    return sc_pass


_sc_pass1 = _make_sc_pass(with_deg=True)
_sc_pass2 = _make_sc_pass(with_deg=False)

_zmesh = plsc.VectorSubcoreMesh(core_axis_name="c", subcore_axis_name="s")


@functools.partial(
    pl.kernel,
    out_type=jax.ShapeDtypeStruct((N_SUB, H2), jnp.float32),
    mesh=_zmesh,
    scratch_types=[
        pltpu.VMEM((ZPT,), jnp.int32),
        pltpu.VMEM((ZPT, H2), jnp.float32),
        pltpu.SemaphoreType.DMA,
    ],
    compiler_params=pltpu.CompilerParams(use_tc_tiling_on_sc=False),
)
def _sc_zgather(mu_hbm, rel_hbm, out_hbm, idx_v, rows_v, sem):
    base = (lax.axis_index("c") * NS + lax.axis_index("s")) * ZPT
    pltpu.sync_copy(rel_hbm.at[pl.ds(base, ZPT)], idx_v)
    pltpu.async_copy(mu_hbm.at[idx_v], rows_v, sem).wait()
    pltpu.sync_copy(rows_v, out_hbm.at[pl.ds(base, ZPT)])


_BLK = 1024


def _mm1(xp, W1):
    def body(x_ref, w_ref, o_ref):
        o_ref[...] = jnp.dot(x_ref[...], w_ref[...],
                             preferred_element_type=jnp.float32)

    return pl.pallas_call(
        body,
        grid=(NPAD // _BLK,),
        in_specs=[pl.BlockSpec((_BLK, D_IN), lambda i: (i, 0)),
                  pl.BlockSpec((D_IN, H1), lambda i: (0, 0))],
        out_specs=pl.BlockSpec((_BLK, H1), lambda i: (i, 0)),
        out_shape=jax.ShapeDtypeStruct((NPAD, H1), jnp.float32),
    )(xp, W1)


def _merge1(p0, p1, yt, d0, d1, b1_2d):
    def body(p0_ref, p1_ref, yt_ref, d0_ref, d1_ref, b_ref, h_ref, inv_ref):
        den = d0_ref[:, 0:1] + d1_ref[:, 0:1] + 1.0
        inv = 1.0 / den
        num = p0_ref[...] + p1_ref[...] - yt_ref[...]
        h_ref[...] = jnp.maximum(num * inv + b_ref[0:1, :], 0.0)
        inv_ref[...] = inv

    return pl.pallas_call(
        body,
        grid=(NPAD // _BLK,),
        in_specs=[pl.BlockSpec((_BLK, H1), lambda i: (i, 0)),
                  pl.BlockSpec((_BLK, H1), lambda i: (i, 0)),
                  pl.BlockSpec((_BLK, H1), lambda i: (i, 0)),
                  pl.BlockSpec((_BLK, DW), lambda i: (i, 0)),
                  pl.BlockSpec((_BLK, DW), lambda i: (i, 0)),
                  pl.BlockSpec((8, H1), lambda i: (0, 0))],
        out_specs=[pl.BlockSpec((_BLK, H1), lambda i: (i, 0)),
                   pl.BlockSpec((_BLK, 1), lambda i: (i, 0))],
        out_shape=[jax.ShapeDtypeStruct((NPAD, H1), jnp.float32),
                   jax.ShapeDtypeStruct((NPAD, 1), jnp.float32)],
    )(p0, p1, yt, d0, d1, b1_2d)


def _heads(q0, q1, h1, inv, W2, b2_2d, W3, b3_2d):
    def body(q0_ref, q1_ref, h_ref, inv_ref, w2_ref, b2_ref, w3_ref, b3_ref,
             mu_ref, lv_ref):
        nbar = (q0_ref[...] + q1_ref[...] - h_ref[...]) * inv_ref[...]
        mu_ref[...] = jnp.dot(nbar, w2_ref[...],
                              preferred_element_type=jnp.float32) + b2_ref[0:1, :]
        lv_ref[...] = jnp.dot(nbar, w3_ref[...],
                              preferred_element_type=jnp.float32) + b3_ref[0:1, :]

    return pl.pallas_call(
        body,
        grid=(NPAD // _BLK,),
        in_specs=[pl.BlockSpec((_BLK, H1), lambda i: (i, 0)),
                  pl.BlockSpec((_BLK, H1), lambda i: (i, 0)),
                  pl.BlockSpec((_BLK, H1), lambda i: (i, 0)),
                  pl.BlockSpec((_BLK, 1), lambda i: (i, 0)),
                  pl.BlockSpec((H1, H2), lambda i: (0, 0)),
                  pl.BlockSpec((8, H2), lambda i: (0, 0)),
                  pl.BlockSpec((H1, H2), lambda i: (0, 0)),
                  pl.BlockSpec((8, H2), lambda i: (0, 0))],
        out_specs=[pl.BlockSpec((_BLK, H2), lambda i: (i, 0)),
                   pl.BlockSpec((_BLK, H2), lambda i: (i, 0))],
        out_shape=[jax.ShapeDtypeStruct((NPAD, H2), jnp.float32),
                   jax.ShapeDtypeStruct((NPAD, H2), jnp.float32)],
    )(q0, q1, h1, inv, W2, b2_2d, W3, b3_2d)


def _decode(z):
    def body(z_ref, o_ref):
        zz = z_ref[...]
        o_ref[...] = lax.dot_general(zz, zz, (((1,), (1,)), ((), ())),
                                     preferred_element_type=jnp.float32)

    return pl.pallas_call(
        body,
        out_shape=jax.ShapeDtypeStruct((N_SUB, N_SUB), jnp.float32),
    )(z)


def kernel(features, edge_index, relative_node_idx, W1, b1, W2, b2, W3, b3):
    src = edge_index[0]
    dst = edge_index[1]
    epad = EPAD - N_EDGES
    src_p = jnp.concatenate([src, jnp.zeros((epad,), jnp.int32)])
    dst_p = jnp.concatenate([dst, jnp.full((epad,), NPAD - 1, jnp.int32)])
    xp = jnp.pad(features, ((0, NPAD - N_NODES), (0, 0)))
    b1_2d = jnp.broadcast_to(b1, (8, H1))
    b2_2d = jnp.broadcast_to(b2, (8, H2))
    b3_2d = jnp.broadcast_to(b3, (8, H2))

    yt = _mm1(xp, W1)
    p, degp = _sc_pass1(yt, src_p, dst_p)
    h1, inv = _merge1(p[0], p[1], yt, degp[0], degp[1], b1_2d)
    q, = _sc_pass2(h1, src_p, dst_p)
    mu_full, lv_full = _heads(q[0], q[1], h1, inv, W2, b2_2d, W3, b3_2d)
    z = _sc_zgather(mu_full, relative_node_idx)
    recovered = _decode(z)
    return recovered, mu_full[:N_NODES], lv_full[:N_NODES]


# speedup vs baseline: 8.9862x; 1.4482x over previous
import functools

import jax
import jax.numpy as jnp
from jax import lax
from jax.experimental import pallas as pl
from jax.experimental.pallas import tpu as pltpu
from jax.experimental.pallas import tpu_sc as plsc

N_NODES = 10000
N_EDGES = 320000
D_IN = 128
H1 = 32
H2 = 16
N_SUB = 1024

NC = 2
NS = 16
NW = NC * NS
L = 16

NPAD = 10240
EPAD = NW * 10240
EPT = EPAD // NW
ECHUNK = 128
NCHUNK = EPT // ECHUNK
RPT = NPAD // NS
ZPT = N_SUB // NW


DW = 16
KPIPE = 8
NSUPER = NCHUNK // KPIPE


def _make_sc_pass(with_deg):
    mesh = plsc.VectorSubcoreMesh(core_axis_name="c", subcore_axis_name="s")
    out_type = [jax.ShapeDtypeStruct((NC, NPAD, H1), jnp.float32)]
    scratch = [
        pltpu.VMEM_SHARED((NPAD, H1), jnp.float32),
        pltpu.VMEM((NCHUNK, ECHUNK), jnp.int32),
        pltpu.VMEM((NCHUNK, ECHUNK), jnp.int32),
        pltpu.VMEM((KPIPE, ECHUNK, H1), jnp.float32),
        pltpu.VMEM((RPT, H1), jnp.float32),
        pltpu.SemaphoreType.DMA,
        pltpu.SemaphoreType.DMA,
    ]
    if with_deg:
        out_type = out_type + [jax.ShapeDtypeStruct((NC, NPAD, DW), jnp.float32)]
        scratch = scratch + [
            pltpu.VMEM_SHARED((NPAD, DW), jnp.float32),
            pltpu.VMEM((ECHUNK, DW), jnp.float32),
            pltpu.VMEM((RPT, DW), jnp.float32),
        ]

    @functools.partial(
        pl.kernel, out_type=tuple(out_type), mesh=mesh, scratch_types=scratch,
        compiler_params=pltpu.CompilerParams(use_tc_tiling_on_sc=False))
    def sc_pass(table_hbm, src_hbm, dst_hbm, out_hbm, *rest):
        if with_deg:
            (degp_hbm, acc_sh, src_v, dst_v, rows_v, stage_v, gsem, ssem,
             deg_sh, ones_v, dstage_v) = rest
        else:
            acc_sh, src_v, dst_v, rows_v, stage_v, gsem, ssem = rest
        cid = lax.axis_index("c")
        sid = lax.axis_index("s")
        rbase = sid * RPT
        tbase = (cid * NS + sid) * NCHUNK
        pltpu.sync_copy(src_hbm.at[pl.ds(tbase, NCHUNK)], src_v)
        pltpu.sync_copy(dst_hbm.at[pl.ds(tbase, NCHUNK)], dst_v)
        pltpu.sync_copy(table_hbm.at[pl.ds(rbase, RPT)], stage_v)
        pltpu.sync_copy(stage_v, acc_sh.at[pl.ds(rbase, RPT)])

        if with_deg:
            ones16 = jnp.full((L,), 1.0, jnp.float32)
            zeros16 = jnp.zeros((L,), jnp.float32)

            def floop(i, c):
                ones_v[i, :] = ones16
                return c

            lax.fori_loop(0, ECHUNK, floop, 0)

            def zloop(i, c):
                dstage_v[i, :] = zeros16
                return c

            lax.fori_loop(0, RPT, zloop, 0)
            pltpu.sync_copy(dstage_v, deg_sh.at[pl.ds(rbase, RPT)])
        plsc.subcore_barrier()

        def superchunk(s, carry):
            c0 = s * KPIPE
            for j in range(KPIPE):
                pltpu.async_copy(table_hbm.at[src_v.at[c0 + j]],
                                 rows_v.at[j], gsem)
            for j in range(KPIPE):
                pltpu.make_async_copy(table_hbm.at[src_v.at[c0 + j]],
                                      rows_v.at[j], gsem).wait()
            for j in range(KPIPE):
                pltpu.async_copy(rows_v.at[j], acc_sh.at[dst_v.at[c0 + j]],
                                 ssem, add=True)
                if with_deg:
                    pltpu.async_copy(ones_v, deg_sh.at[dst_v.at[c0 + j]],
                                     ssem, add=True)
            for j in range(KPIPE):
                pltpu.make_async_copy(rows_v.at[j], acc_sh.at[dst_v.at[c0 + j]],
                                      ssem).wait()
                if with_deg:
                    pltpu.make_async_copy(ones_v, deg_sh.at[dst_v.at[c0 + j]],
                                          ssem).wait()
            return carry

        lax.fori_loop(0, NSUPER, superchunk, 0)
        plsc.subcore_barrier()
        pltpu.sync_copy(acc_sh.at[pl.ds(rbase, RPT)], stage_v)
        pltpu.sync_copy(stage_v, out_hbm.at[cid].at[pl.ds(rbase, RPT)])
        if with_deg:
            pltpu.sync_copy(deg_sh.at[pl.ds(rbase, RPT)], dstage_v)
            pltpu.sync_copy(dstage_v, degp_hbm.at[cid].at[pl.ds(rbase, RPT)])

    return sc_pass


_sc_pass1 = _make_sc_pass(with_deg=True)
_sc_pass2 = _make_sc_pass(with_deg=False)

_zmesh = plsc.VectorSubcoreMesh(core_axis_name="c", subcore_axis_name="s")


@functools.partial(
    pl.kernel,
    out_type=jax.ShapeDtypeStruct((N_SUB, H2), jnp.float32),
    mesh=_zmesh,
    scratch_types=[
        pltpu.VMEM((ZPT,), jnp.int32),
        pltpu.VMEM((ZPT, H2), jnp.float32),
        pltpu.SemaphoreType.DMA,
    ],
    compiler_params=pltpu.CompilerParams(use_tc_tiling_on_sc=False),
)
def _sc_zgather(mu_hbm, rel_hbm, out_hbm, idx_v, rows_v, sem):
    base = (lax.axis_index("c") * NS + lax.axis_index("s")) * ZPT
    pltpu.sync_copy(rel_hbm.at[pl.ds(base, ZPT)], idx_v)
    pltpu.async_copy(mu_hbm.at[idx_v], rows_v, sem).wait()
    pltpu.sync_copy(rows_v, out_hbm.at[pl.ds(base, ZPT)])


_BLK = 1024


def _mm1(xp, W1):
    def body(x_ref, w_ref, o_ref):
        o_ref[...] = jnp.dot(x_ref[...], w_ref[...],
                             preferred_element_type=jnp.float32)

    return pl.pallas_call(
        body,
        grid=(NPAD // _BLK,),
        in_specs=[pl.BlockSpec((_BLK, D_IN), lambda i: (i, 0)),
                  pl.BlockSpec((D_IN, H1), lambda i: (0, 0))],
        out_specs=pl.BlockSpec((_BLK, H1), lambda i: (i, 0)),
        out_shape=jax.ShapeDtypeStruct((NPAD, H1), jnp.float32),
    )(xp, W1)


def _merge1(p0, p1, yt, d0, d1, b1_2d):
    def body(p0_ref, p1_ref, yt_ref, d0_ref, d1_ref, b_ref, h_ref, inv_ref):
        den = d0_ref[:, 0:1] + d1_ref[:, 0:1] + 1.0
        inv = 1.0 / den
        num = p0_ref[...] + p1_ref[...] - yt_ref[...]
        h_ref[...] = jnp.maximum(num * inv + b_ref[0:1, :], 0.0)
        inv_ref[...] = inv

    return pl.pallas_call(
        body,
        grid=(NPAD // _BLK,),
        in_specs=[pl.BlockSpec((_BLK, H1), lambda i: (i, 0)),
                  pl.BlockSpec((_BLK, H1), lambda i: (i, 0)),
                  pl.BlockSpec((_BLK, H1), lambda i: (i, 0)),
                  pl.BlockSpec((_BLK, DW), lambda i: (i, 0)),
                  pl.BlockSpec((_BLK, DW), lambda i: (i, 0)),
                  pl.BlockSpec((8, H1), lambda i: (0, 0))],
        out_specs=[pl.BlockSpec((_BLK, H1), lambda i: (i, 0)),
                   pl.BlockSpec((_BLK, 1), lambda i: (i, 0))],
        out_shape=[jax.ShapeDtypeStruct((NPAD, H1), jnp.float32),
                   jax.ShapeDtypeStruct((NPAD, 1), jnp.float32)],
    )(p0, p1, yt, d0, d1, b1_2d)


def _heads(q0, q1, h1, inv, W2, b2_2d, W3, b3_2d):
    def body(q0_ref, q1_ref, h_ref, inv_ref, w2_ref, b2_ref, w3_ref, b3_ref,
             mu_ref, lv_ref):
        nbar = (q0_ref[...] + q1_ref[...] - h_ref[...]) * inv_ref[...]
        mu_ref[...] = jnp.dot(nbar, w2_ref[...],
                              preferred_element_type=jnp.float32) + b2_ref[0:1, :]
        lv_ref[...] = jnp.dot(nbar, w3_ref[...],
                              preferred_element_type=jnp.float32) + b3_ref[0:1, :]

    return pl.pallas_call(
        body,
        grid=(NPAD // _BLK,),
        in_specs=[pl.BlockSpec((_BLK, H1), lambda i: (i, 0)),
                  pl.BlockSpec((_BLK, H1), lambda i: (i, 0)),
                  pl.BlockSpec((_BLK, H1), lambda i: (i, 0)),
                  pl.BlockSpec((_BLK, 1), lambda i: (i, 0)),
                  pl.BlockSpec((H1, H2), lambda i: (0, 0)),
                  pl.BlockSpec((8, H2), lambda i: (0, 0)),
                  pl.BlockSpec((H1, H2), lambda i: (0, 0)),
                  pl.BlockSpec((8, H2), lambda i: (0, 0))],
        out_specs=[pl.BlockSpec((_BLK, H2), lambda i: (i, 0)),
                   pl.BlockSpec((_BLK, H2), lambda i: (i, 0))],
        out_shape=[jax.ShapeDtypeStruct((NPAD, H2), jnp.float32),
                   jax.ShapeDtypeStruct((NPAD, H2), jnp.float32)],
    )(q0, q1, h1, inv, W2, b2_2d, W3, b3_2d)


def _decode(z):
    def body(z_ref, o_ref):
        zz = z_ref[...]
        o_ref[...] = lax.dot_general(zz, zz, (((1,), (1,)), ((), ())),
                                     preferred_element_type=jnp.float32)

    return pl.pallas_call(
        body,
        out_shape=jax.ShapeDtypeStruct((N_SUB, N_SUB), jnp.float32),
    )(z)


def kernel(features, edge_index, relative_node_idx, W1, b1, W2, b2, W3, b3):
    src = edge_index[0]
    dst = edge_index[1]
    epad = EPAD - N_EDGES
    src_p = jnp.concatenate([src, jnp.zeros((epad,), jnp.int32)]
                            ).reshape(EPAD // ECHUNK, ECHUNK)
    dst_p = jnp.concatenate([dst, jnp.full((epad,), NPAD - 1, jnp.int32)]
                            ).reshape(EPAD // ECHUNK, ECHUNK)
    xp = jnp.pad(features, ((0, NPAD - N_NODES), (0, 0)))
    b1_2d = jnp.broadcast_to(b1, (8, H1))
    b2_2d = jnp.broadcast_to(b2, (8, H2))
    b3_2d = jnp.broadcast_to(b3, (8, H2))

    yt = _mm1(xp, W1)
    p, degp = _sc_pass1(yt, src_p, dst_p)
    h1, inv = _merge1(p[0], p[1], yt, degp[0], degp[1], b1_2d)
    q, = _sc_pass2(h1, src_p, dst_p)
    mu_full, lv_full = _heads(q[0], q[1], h1, inv, W2, b2_2d, W3, b3_2d)
    z = _sc_zgather(mu_full, relative_node_idx)
    recovered = _decode(z)
    return recovered, mu_full[:N_NODES], lv_full[:N_NODES]


# speedup vs baseline: 9.0813x; 1.0106x over previous
import functools

import jax
import jax.numpy as jnp
from jax import lax
from jax.experimental import pallas as pl
from jax.experimental.pallas import tpu as pltpu
from jax.experimental.pallas import tpu_sc as plsc

N_NODES = 10000
N_EDGES = 320000
D_IN = 128
H1 = 32
H2 = 16
N_SUB = 1024

NC = 2
NS = 16
NW = NC * NS
L = 16

NPAD = 10240
EPAD = NW * 10240
EPT = EPAD // NW
ECHUNK = 128
NCHUNK = EPT // ECHUNK
RPT = NPAD // NS
ZPT = N_SUB // NW


DW = 16
KPIPE = 8
NSUPER = NCHUNK // KPIPE


def _make_sc_pass(with_deg):
    mesh = plsc.VectorSubcoreMesh(core_axis_name="c", subcore_axis_name="s")
    out_type = [jax.ShapeDtypeStruct((NC, NPAD, H1), jnp.float32)]
    scratch = [
        pltpu.VMEM_SHARED((NPAD, H1), jnp.float32),
        pltpu.VMEM((NCHUNK, ECHUNK), jnp.int32),
        pltpu.VMEM((NCHUNK, ECHUNK), jnp.int32),
        pltpu.VMEM((KPIPE, ECHUNK, H1), jnp.float32),
        pltpu.VMEM((RPT, H1), jnp.float32),
        pltpu.SemaphoreType.DMA,
        pltpu.SemaphoreType.DMA,
    ]
    if with_deg:
        out_type = out_type + [jax.ShapeDtypeStruct((NC, NPAD, DW), jnp.float32)]
        scratch = scratch + [
            pltpu.VMEM_SHARED((NPAD, DW), jnp.float32),
            pltpu.VMEM((ECHUNK, DW), jnp.float32),
            pltpu.VMEM((RPT, DW), jnp.float32),
        ]

    @functools.partial(
        pl.kernel, out_type=tuple(out_type), mesh=mesh, scratch_types=scratch,
        compiler_params=pltpu.CompilerParams(use_tc_tiling_on_sc=False))
    def sc_pass(table_hbm, src_hbm, dst_hbm, out_hbm, *rest):
        if with_deg:
            (degp_hbm, acc_sh, src_v, dst_v, rows_v, stage_v, gsem, ssem,
             deg_sh, ones_v, dstage_v) = rest
        else:
            acc_sh, src_v, dst_v, rows_v, stage_v, gsem, ssem = rest
        cid = lax.axis_index("c")
        sid = lax.axis_index("s")
        rbase = sid * RPT
        tbase = (cid * NS + sid) * NCHUNK
        pltpu.sync_copy(src_hbm.at[pl.ds(tbase, NCHUNK)], src_v)
        pltpu.sync_copy(dst_hbm.at[pl.ds(tbase, NCHUNK)], dst_v)
        pltpu.sync_copy(table_hbm.at[pl.ds(rbase, RPT)], stage_v)
        pltpu.sync_copy(stage_v, acc_sh.at[pl.ds(rbase, RPT)])

        if with_deg:
            ones16 = jnp.full((L,), 1.0, jnp.float32)
            zeros16 = jnp.zeros((L,), jnp.float32)

            def floop(i, c):
                ones_v[i, :] = ones16
                return c

            lax.fori_loop(0, ECHUNK, floop, 0)

            def zloop(i, c):
                dstage_v[i, :] = zeros16
                return c

            lax.fori_loop(0, RPT, zloop, 0)
            pltpu.sync_copy(dstage_v, deg_sh.at[pl.ds(rbase, RPT)])
        plsc.subcore_barrier()

        def superchunk(s, carry):
            c0 = s * KPIPE
            for j in range(KPIPE):
                pltpu.async_copy(table_hbm.at[src_v.at[c0 + j]],
                                 rows_v.at[j], gsem)
            for j in range(KPIPE):
                pltpu.make_async_copy(table_hbm.at[src_v.at[c0 + j]],
                                      rows_v.at[j], gsem).wait()
            for j in range(KPIPE):
                pltpu.async_copy(rows_v.at[j], acc_sh.at[dst_v.at[c0 + j]],
                                 ssem, add=True)
                if with_deg:
                    pltpu.async_copy(ones_v, deg_sh.at[dst_v.at[c0 + j]],
                                     ssem, add=True)
            for j in range(KPIPE):
                pltpu.make_async_copy(rows_v.at[j], acc_sh.at[dst_v.at[c0 + j]],
                                      ssem).wait()
                if with_deg:
                    pltpu.make_async_copy(ones_v, deg_sh.at[dst_v.at[c0 + j]],
                                          ssem).wait()
            return carry

        lax.fori_loop(0, NSUPER, superchunk, 0)
        plsc.subcore_barrier()
        pltpu.sync_copy(acc_sh.at[pl.ds(rbase, RPT)], stage_v)
        pltpu.sync_copy(stage_v, out_hbm.at[cid].at[pl.ds(rbase, RPT)])
        if with_deg:
            pltpu.sync_copy(deg_sh.at[pl.ds(rbase, RPT)], dstage_v)
            pltpu.sync_copy(dstage_v, degp_hbm.at[cid].at[pl.ds(rbase, RPT)])

    return sc_pass


_sc_pass1 = _make_sc_pass(with_deg=True)
_sc_pass2 = _make_sc_pass(with_deg=False)

_zmesh = plsc.VectorSubcoreMesh(core_axis_name="c", subcore_axis_name="s")


@functools.partial(
    pl.kernel,
    out_type=jax.ShapeDtypeStruct((N_SUB, H2), jnp.float32),
    mesh=_zmesh,
    scratch_types=[
        pltpu.VMEM((ZPT,), jnp.int32),
        pltpu.VMEM((ZPT, H2), jnp.float32),
        pltpu.SemaphoreType.DMA,
    ],
    compiler_params=pltpu.CompilerParams(use_tc_tiling_on_sc=False),
)
def _sc_zgather(mu_hbm, rel_hbm, out_hbm, idx_v, rows_v, sem):
    base = (lax.axis_index("c") * NS + lax.axis_index("s")) * ZPT
    pltpu.sync_copy(rel_hbm.at[pl.ds(base, ZPT)], idx_v)
    pltpu.async_copy(mu_hbm.at[idx_v], rows_v, sem).wait()
    pltpu.sync_copy(rows_v, out_hbm.at[pl.ds(base, ZPT)])


_BLK = 1024


def _mm1(xp, W1):
    def body(x_ref, w_ref, o_ref):
        o_ref[...] = jnp.dot(x_ref[...], w_ref[...],
                             preferred_element_type=jnp.float32)

    return pl.pallas_call(
        body,
        grid=(NPAD // _BLK,),
        in_specs=[pl.BlockSpec((_BLK, D_IN), lambda i: (i, 0)),
                  pl.BlockSpec((D_IN, H1), lambda i: (0, 0))],
        out_specs=pl.BlockSpec((_BLK, H1), lambda i: (i, 0)),
        out_shape=jax.ShapeDtypeStruct((NPAD, H1), jnp.float32),
    )(xp, W1)


def _merge1(p0, p1, yt, d0, d1, b1_2d):
    def body(p0_ref, p1_ref, yt_ref, d0_ref, d1_ref, b_ref, h_ref, inv_ref):
        den = d0_ref[:, 0:1] + d1_ref[:, 0:1] + 1.0
        inv = 1.0 / den
        num = p0_ref[...] + p1_ref[...] - yt_ref[...]
        h_ref[...] = jnp.maximum(num * inv + b_ref[0:1, :], 0.0)
        inv_ref[...] = inv

    return pl.pallas_call(
        body,
        grid=(NPAD // _BLK,),
        in_specs=[pl.BlockSpec((_BLK, H1), lambda i: (i, 0)),
                  pl.BlockSpec((_BLK, H1), lambda i: (i, 0)),
                  pl.BlockSpec((_BLK, H1), lambda i: (i, 0)),
                  pl.BlockSpec((_BLK, DW), lambda i: (i, 0)),
                  pl.BlockSpec((_BLK, DW), lambda i: (i, 0)),
                  pl.BlockSpec((8, H1), lambda i: (0, 0))],
        out_specs=[pl.BlockSpec((_BLK, H1), lambda i: (i, 0)),
                   pl.BlockSpec((_BLK, 1), lambda i: (i, 0))],
        out_shape=[jax.ShapeDtypeStruct((NPAD, H1), jnp.float32),
                   jax.ShapeDtypeStruct((NPAD, 1), jnp.float32)],
    )(p0, p1, yt, d0, d1, b1_2d)


def _heads(q0, q1, h1, inv, W2, b2_2d, W3, b3_2d):
    def body(q0_ref, q1_ref, h_ref, inv_ref, w2_ref, b2_ref, w3_ref, b3_ref,
             mu_ref, lv_ref):
        nbar = (q0_ref[...] + q1_ref[...] - h_ref[...]) * inv_ref[...]
        mu_ref[...] = jnp.dot(nbar, w2_ref[...],
                              preferred_element_type=jnp.float32) + b2_ref[0:1, :]
        lv_ref[...] = jnp.dot(nbar, w3_ref[...],
                              preferred_element_type=jnp.float32) + b3_ref[0:1, :]

    return pl.pallas_call(
        body,
        grid=(NPAD // _BLK,),
        in_specs=[pl.BlockSpec((_BLK, H1), lambda i: (i, 0)),
                  pl.BlockSpec((_BLK, H1), lambda i: (i, 0)),
                  pl.BlockSpec((_BLK, H1), lambda i: (i, 0)),
                  pl.BlockSpec((_BLK, 1), lambda i: (i, 0)),
                  pl.BlockSpec((H1, H2), lambda i: (0, 0)),
                  pl.BlockSpec((8, H2), lambda i: (0, 0)),
                  pl.BlockSpec((H1, H2), lambda i: (0, 0)),
                  pl.BlockSpec((8, H2), lambda i: (0, 0))],
        out_specs=[pl.BlockSpec((_BLK, H2), lambda i: (i, 0)),
                   pl.BlockSpec((_BLK, H2), lambda i: (i, 0))],
        out_shape=[jax.ShapeDtypeStruct((NPAD, H2), jnp.float32),
                   jax.ShapeDtypeStruct((NPAD, H2), jnp.float32)],
    )(q0, q1, h1, inv, W2, b2_2d, W3, b3_2d)


def _decode(z):
    def body(z_ref, o_ref):
        zz = z_ref[...]
        o_ref[...] = lax.dot_general(zz, zz, (((1,), (1,)), ((), ())),
                                     preferred_element_type=jnp.float32)

    return pl.pallas_call(
        body,
        out_shape=jax.ShapeDtypeStruct((N_SUB, N_SUB), jnp.float32),
    )(z)


def kernel(features, edge_index, relative_node_idx, W1, b1, W2, b2, W3, b3):
    src = edge_index[0]
    dst = edge_index[1]
    epad = EPAD - N_EDGES
    src_p = jnp.concatenate([src, jnp.zeros((epad,), jnp.int32)]
                            ).reshape(EPAD // ECHUNK, ECHUNK)
    pad_dst = N_NODES + jnp.arange(epad, dtype=jnp.int32) % (NPAD - N_NODES)
    dst_p = jnp.concatenate([dst, pad_dst]).reshape(EPAD // ECHUNK, ECHUNK)
    xp = jnp.pad(features, ((0, NPAD - N_NODES), (0, 0)))
    b1_2d = jnp.broadcast_to(b1, (8, H1))
    b2_2d = jnp.broadcast_to(b2, (8, H2))
    b3_2d = jnp.broadcast_to(b3, (8, H2))

    yt = _mm1(xp, W1)
    p, degp = _sc_pass1(yt, src_p, dst_p)
    h1, inv = _merge1(p[0], p[1], yt, degp[0], degp[1], b1_2d)
    q, = _sc_pass2(h1, src_p, dst_p)
    mu_full, lv_full = _heads(q[0], q[1], h1, inv, W2, b2_2d, W3, b3_2d)
    z = _sc_zgather(mu_full, relative_node_idx)
    recovered = _decode(z)
    return recovered, mu_full[:N_NODES], lv_full[:N_NODES]


# speedup vs baseline: 13.8989x; 1.5305x over previous
import functools

import jax
import jax.numpy as jnp
from jax import lax
from jax.experimental import pallas as pl
from jax.experimental.pallas import tpu as pltpu
from jax.experimental.pallas import tpu_sc as plsc

N_NODES = 10000
N_EDGES = 320000
D_IN = 128
H1 = 32
H2 = 16
N_SUB = 1024

NC = 2
NS = 16
NW = NC * NS
L = 16

NPAD = 10240
EPAD = NW * 10240
EPT = EPAD // NW
ECHUNK = 128
NCHUNK = EPT // ECHUNK
RPT = NPAD // NS
ZPT = N_SUB // NW


DW = 16
KPIPE = 8
NSUPER = NCHUNK // KPIPE


def _make_sc_pass(with_deg):
    mesh = plsc.VectorSubcoreMesh(core_axis_name="c", subcore_axis_name="s")
    out_type = [jax.ShapeDtypeStruct((NC, NPAD, H1), jnp.float32)]
    scratch = [
        pltpu.VMEM_SHARED((NPAD, H1), jnp.float32),
        pltpu.VMEM_SHARED((NPAD, H1), jnp.float32),
        pltpu.VMEM((NCHUNK, ECHUNK), jnp.int32),
        pltpu.VMEM((NCHUNK, ECHUNK), jnp.int32),
        pltpu.VMEM((KPIPE, ECHUNK, H1), jnp.float32),
        pltpu.SemaphoreType.DMA,
        pltpu.SemaphoreType.DMA,
    ]
    if with_deg:
        out_type = out_type + [jax.ShapeDtypeStruct((NC, NPAD, DW), jnp.float32)]
        scratch = scratch + [
            pltpu.VMEM_SHARED((NPAD, DW), jnp.float32),
            pltpu.VMEM((ECHUNK, DW), jnp.float32),
        ]

    @functools.partial(
        pl.kernel, out_type=tuple(out_type), mesh=mesh, scratch_types=scratch,
        compiler_params=pltpu.CompilerParams(use_tc_tiling_on_sc=False))
    def sc_pass(table_hbm, src_hbm, dst_hbm, out_hbm, *rest):
        if with_deg:
            (degp_hbm, acc_sh, tab_sh, src_v, dst_v, rows_v, gsem,
             ssem, deg_sh, ones_v) = rest
        else:
            acc_sh, tab_sh, src_v, dst_v, rows_v, gsem, ssem = rest
        cid = lax.axis_index("c")
        sid = lax.axis_index("s")
        rbase = sid * RPT
        tbase = (cid * NS + sid) * NCHUNK
        pltpu.sync_copy(src_hbm.at[pl.ds(tbase, NCHUNK)], src_v)
        pltpu.sync_copy(dst_hbm.at[pl.ds(tbase, NCHUNK)], dst_v)
        for k in range(RPT // ECHUNK):
            r0 = rbase + k * ECHUNK
            pltpu.sync_copy(table_hbm.at[pl.ds(r0, ECHUNK)], rows_v.at[0])
            pltpu.sync_copy(rows_v.at[0], acc_sh.at[pl.ds(r0, ECHUNK)])
            pltpu.sync_copy(rows_v.at[0], tab_sh.at[pl.ds(r0, ECHUNK)])

        if with_deg:
            ones16 = jnp.full((L,), 1.0, jnp.float32)
            zeros16 = jnp.zeros((L,), jnp.float32)

            def zloop(i, c):
                ones_v[i, :] = zeros16
                return c

            lax.fori_loop(0, ECHUNK, zloop, 0)
            for k in range(RPT // ECHUNK):
                pltpu.sync_copy(ones_v, deg_sh.at[pl.ds(rbase + k * ECHUNK,
                                                        ECHUNK)])

            def floop(i, c):
                ones_v[i, :] = ones16
                return c

            lax.fori_loop(0, ECHUNK, floop, 0)
        plsc.subcore_barrier()

        def superchunk(s, carry):
            c0 = s * KPIPE
            for j in range(KPIPE):
                pltpu.async_copy(tab_sh.at[src_v.at[c0 + j]],
                                 rows_v.at[j], gsem)
            for j in range(KPIPE):
                pltpu.make_async_copy(tab_sh.at[src_v.at[c0 + j]],
                                      rows_v.at[j], gsem).wait()
            for j in range(KPIPE):
                pltpu.async_copy(rows_v.at[j], acc_sh.at[dst_v.at[c0 + j]],
                                 ssem, add=True)
                if with_deg:
                    pltpu.async_copy(ones_v, deg_sh.at[dst_v.at[c0 + j]],
                                     ssem, add=True)
            for j in range(KPIPE):
                pltpu.make_async_copy(rows_v.at[j], acc_sh.at[dst_v.at[c0 + j]],
                                      ssem).wait()
                if with_deg:
                    pltpu.make_async_copy(ones_v, deg_sh.at[dst_v.at[c0 + j]],
                                          ssem).wait()
            return carry

        lax.fori_loop(0, NSUPER, superchunk, 0)
        plsc.subcore_barrier()
        for k in range(RPT // ECHUNK):
            r0 = rbase + k * ECHUNK
            pltpu.sync_copy(acc_sh.at[pl.ds(r0, ECHUNK)], rows_v.at[0])
            pltpu.sync_copy(rows_v.at[0], out_hbm.at[cid].at[pl.ds(r0, ECHUNK)])
        if with_deg:
            for k in range(RPT // ECHUNK):
                r0 = rbase + k * ECHUNK
                pltpu.sync_copy(deg_sh.at[pl.ds(r0, ECHUNK)], ones_v)
                pltpu.sync_copy(ones_v, degp_hbm.at[cid].at[pl.ds(r0, ECHUNK)])

    return sc_pass


_sc_pass1 = _make_sc_pass(with_deg=True)
_sc_pass2 = _make_sc_pass(with_deg=False)

_zmesh = plsc.VectorSubcoreMesh(core_axis_name="c", subcore_axis_name="s")


@functools.partial(
    pl.kernel,
    out_type=jax.ShapeDtypeStruct((N_SUB, H2), jnp.float32),
    mesh=_zmesh,
    scratch_types=[
        pltpu.VMEM((ZPT,), jnp.int32),
        pltpu.VMEM((ZPT, H2), jnp.float32),
        pltpu.SemaphoreType.DMA,
    ],
    compiler_params=pltpu.CompilerParams(use_tc_tiling_on_sc=False),
)
def _sc_zgather(mu_hbm, rel_hbm, out_hbm, idx_v, rows_v, sem):
    base = (lax.axis_index("c") * NS + lax.axis_index("s")) * ZPT
    pltpu.sync_copy(rel_hbm.at[pl.ds(base, ZPT)], idx_v)
    pltpu.async_copy(mu_hbm.at[idx_v], rows_v, sem).wait()
    pltpu.sync_copy(rows_v, out_hbm.at[pl.ds(base, ZPT)])


_BLK = 1024


def _mm1(xp, W1):
    def body(x_ref, w_ref, o_ref):
        o_ref[...] = jnp.dot(x_ref[...], w_ref[...],
                             preferred_element_type=jnp.float32)

    return pl.pallas_call(
        body,
        grid=(NPAD // _BLK,),
        in_specs=[pl.BlockSpec((_BLK, D_IN), lambda i: (i, 0)),
                  pl.BlockSpec((D_IN, H1), lambda i: (0, 0))],
        out_specs=pl.BlockSpec((_BLK, H1), lambda i: (i, 0)),
        out_shape=jax.ShapeDtypeStruct((NPAD, H1), jnp.float32),
    )(xp, W1)


def _merge1(p0, p1, yt, d0, d1, b1_2d):
    def body(p0_ref, p1_ref, yt_ref, d0_ref, d1_ref, b_ref, h_ref, inv_ref):
        den = d0_ref[:, 0:1] + d1_ref[:, 0:1] + 1.0
        inv = 1.0 / den
        num = p0_ref[...] + p1_ref[...] - yt_ref[...]
        h_ref[...] = jnp.maximum(num * inv + b_ref[0:1, :], 0.0)
        inv_ref[...] = inv

    return pl.pallas_call(
        body,
        grid=(NPAD // _BLK,),
        in_specs=[pl.BlockSpec((_BLK, H1), lambda i: (i, 0)),
                  pl.BlockSpec((_BLK, H1), lambda i: (i, 0)),
                  pl.BlockSpec((_BLK, H1), lambda i: (i, 0)),
                  pl.BlockSpec((_BLK, DW), lambda i: (i, 0)),
                  pl.BlockSpec((_BLK, DW), lambda i: (i, 0)),
                  pl.BlockSpec((8, H1), lambda i: (0, 0))],
        out_specs=[pl.BlockSpec((_BLK, H1), lambda i: (i, 0)),
                   pl.BlockSpec((_BLK, 1), lambda i: (i, 0))],
        out_shape=[jax.ShapeDtypeStruct((NPAD, H1), jnp.float32),
                   jax.ShapeDtypeStruct((NPAD, 1), jnp.float32)],
    )(p0, p1, yt, d0, d1, b1_2d)


def _heads(q0, q1, h1, inv, W2, b2_2d, W3, b3_2d):
    def body(q0_ref, q1_ref, h_ref, inv_ref, w2_ref, b2_ref, w3_ref, b3_ref,
             mu_ref, lv_ref):
        nbar = (q0_ref[...] + q1_ref[...] - h_ref[...]) * inv_ref[...]
        mu_ref[...] = jnp.dot(nbar, w2_ref[...],
                              preferred_element_type=jnp.float32) + b2_ref[0:1, :]
        lv_ref[...] = jnp.dot(nbar, w3_ref[...],
                              preferred_element_type=jnp.float32) + b3_ref[0:1, :]

    return pl.pallas_call(
        body,
        grid=(NPAD // _BLK,),
        in_specs=[pl.BlockSpec((_BLK, H1), lambda i: (i, 0)),
                  pl.BlockSpec((_BLK, H1), lambda i: (i, 0)),
                  pl.BlockSpec((_BLK, H1), lambda i: (i, 0)),
                  pl.BlockSpec((_BLK, 1), lambda i: (i, 0)),
                  pl.BlockSpec((H1, H2), lambda i: (0, 0)),
                  pl.BlockSpec((8, H2), lambda i: (0, 0)),
                  pl.BlockSpec((H1, H2), lambda i: (0, 0)),
                  pl.BlockSpec((8, H2), lambda i: (0, 0))],
        out_specs=[pl.BlockSpec((_BLK, H2), lambda i: (i, 0)),
                   pl.BlockSpec((_BLK, H2), lambda i: (i, 0))],
        out_shape=[jax.ShapeDtypeStruct((NPAD, H2), jnp.float32),
                   jax.ShapeDtypeStruct((NPAD, H2), jnp.float32)],
    )(q0, q1, h1, inv, W2, b2_2d, W3, b3_2d)


def _decode(z):
    def body(z_ref, o_ref):
        zz = z_ref[...]
        o_ref[...] = lax.dot_general(zz, zz, (((1,), (1,)), ((), ())),
                                     preferred_element_type=jnp.float32)

    return pl.pallas_call(
        body,
        out_shape=jax.ShapeDtypeStruct((N_SUB, N_SUB), jnp.float32),
    )(z)


def kernel(features, edge_index, relative_node_idx, W1, b1, W2, b2, W3, b3):
    src = edge_index[0]
    dst = edge_index[1]
    epad = EPAD - N_EDGES
    src_p = jnp.concatenate([src, jnp.zeros((epad,), jnp.int32)]
                            ).reshape(EPAD // ECHUNK, ECHUNK)
    pad_dst = N_NODES + jnp.arange(epad, dtype=jnp.int32) % (NPAD - N_NODES)
    dst_p = jnp.concatenate([dst, pad_dst]).reshape(EPAD // ECHUNK, ECHUNK)
    xp = jnp.pad(features, ((0, NPAD - N_NODES), (0, 0)))
    b1_2d = jnp.broadcast_to(b1, (8, H1))
    b2_2d = jnp.broadcast_to(b2, (8, H2))
    b3_2d = jnp.broadcast_to(b3, (8, H2))

    yt = _mm1(xp, W1)
    p, degp = _sc_pass1(yt, src_p, dst_p)
    h1, inv = _merge1(p[0], p[1], yt, degp[0], degp[1], b1_2d)
    q, = _sc_pass2(h1, src_p, dst_p)
    mu_full, lv_full = _heads(q[0], q[1], h1, inv, W2, b2_2d, W3, b3_2d)
    z = _sc_zgather(mu_full, relative_node_idx)
    recovered = _decode(z)
    return recovered, mu_full[:N_NODES], lv_full[:N_NODES]


# speedup vs baseline: 14.2231x; 1.0233x over previous
import functools

import jax
import jax.numpy as jnp
from jax import lax
from jax.experimental import pallas as pl
from jax.experimental.pallas import tpu as pltpu
from jax.experimental.pallas import tpu_sc as plsc

N_NODES = 10000
N_EDGES = 320000
D_IN = 128
H1 = 32
H2 = 16
N_SUB = 1024

NC = 2
NS = 16
NW = NC * NS
L = 16

NPAD = 10240
EPAD = NW * 10240
EPT = EPAD // NW
ECHUNK = 128
NCHUNK = EPT // ECHUNK
RPT = NPAD // NS
ZPT = N_SUB // NW


DW = 16
KPIPE = 8
NSUPER = NCHUNK // KPIPE


def _make_sc_pass(with_deg):
    mesh = plsc.VectorSubcoreMesh(core_axis_name="c", subcore_axis_name="s")
    out_type = [jax.ShapeDtypeStruct((NC, NPAD, H1), jnp.float32)]
    scratch = [
        pltpu.VMEM_SHARED((NPAD, H1), jnp.float32),
        pltpu.VMEM_SHARED((NPAD, H1), jnp.float32),
        pltpu.VMEM((NCHUNK, ECHUNK), jnp.int32),
        pltpu.VMEM((NCHUNK, ECHUNK), jnp.int32),
        pltpu.VMEM((KPIPE, ECHUNK, H1), jnp.float32),
        pltpu.SemaphoreType.DMA,
        pltpu.SemaphoreType.DMA,
    ]
    if with_deg:
        out_type = out_type + [jax.ShapeDtypeStruct((NC, NPAD, DW), jnp.float32)]
        scratch = scratch + [
            pltpu.VMEM_SHARED((NPAD, DW), jnp.float32),
            pltpu.VMEM((ECHUNK, DW), jnp.float32),
        ]

    @functools.partial(
        pl.kernel, out_type=tuple(out_type), mesh=mesh, scratch_types=scratch,
        compiler_params=pltpu.CompilerParams(use_tc_tiling_on_sc=False))
    def sc_pass(table_hbm, src_hbm, dst_hbm, out_hbm, *rest):
        if with_deg:
            (degp_hbm, acc_sh, tab_sh, src_v, dst_v, rows_v, gsem,
             ssem, deg_sh, ones_v) = rest
        else:
            acc_sh, tab_sh, src_v, dst_v, rows_v, gsem, ssem = rest
        cid = lax.axis_index("c")
        sid = lax.axis_index("s")
        rbase = sid * RPT
        tbase = (cid * NS + sid) * NCHUNK
        pltpu.sync_copy(src_hbm.at[pl.ds(tbase, NCHUNK)], src_v)
        pltpu.sync_copy(dst_hbm.at[pl.ds(tbase, NCHUNK)], dst_v)
        pltpu.sync_copy(table_hbm.at[pl.ds(rbase, RPT)],
                        acc_sh.at[pl.ds(rbase, RPT)])
        pltpu.sync_copy(table_hbm.at[pl.ds(rbase, RPT)],
                        tab_sh.at[pl.ds(rbase, RPT)])

        if with_deg:
            ones16 = jnp.full((L,), 1.0, jnp.float32)
            zeros16 = jnp.zeros((L,), jnp.float32)

            def zloop(i, c):
                ones_v[i, :] = zeros16
                return c

            lax.fori_loop(0, ECHUNK, zloop, 0)
            for k in range(RPT // ECHUNK):
                pltpu.sync_copy(ones_v, deg_sh.at[pl.ds(rbase + k * ECHUNK,
                                                        ECHUNK)])

            def floop(i, c):
                ones_v[i, :] = ones16
                return c

            lax.fori_loop(0, ECHUNK, floop, 0)
        plsc.subcore_barrier()

        def superchunk(s, carry):
            c0 = s * KPIPE
            for j in range(KPIPE):
                pltpu.async_copy(tab_sh.at[src_v.at[c0 + j]],
                                 rows_v.at[j], gsem)
            for j in range(KPIPE):
                pltpu.make_async_copy(tab_sh.at[src_v.at[c0 + j]],
                                      rows_v.at[j], gsem).wait()
            for j in range(KPIPE):
                pltpu.async_copy(rows_v.at[j], acc_sh.at[dst_v.at[c0 + j]],
                                 ssem, add=True)
                if with_deg:
                    pltpu.async_copy(ones_v, deg_sh.at[dst_v.at[c0 + j]],
                                     ssem, add=True)
            for j in range(KPIPE):
                pltpu.make_async_copy(rows_v.at[j], acc_sh.at[dst_v.at[c0 + j]],
                                      ssem).wait()
                if with_deg:
                    pltpu.make_async_copy(ones_v, deg_sh.at[dst_v.at[c0 + j]],
                                          ssem).wait()
            return carry

        lax.fori_loop(0, NSUPER, superchunk, 0)
        plsc.subcore_barrier()
        pltpu.sync_copy(acc_sh.at[pl.ds(rbase, RPT)],
                        out_hbm.at[cid].at[pl.ds(rbase, RPT)])
        if with_deg:
            pltpu.sync_copy(deg_sh.at[pl.ds(rbase, RPT)],
                            degp_hbm.at[cid].at[pl.ds(rbase, RPT)])

    return sc_pass


_sc_pass1 = _make_sc_pass(with_deg=True)
_sc_pass2 = _make_sc_pass(with_deg=False)

_zmesh = plsc.VectorSubcoreMesh(core_axis_name="c", subcore_axis_name="s")


@functools.partial(
    pl.kernel,
    out_type=jax.ShapeDtypeStruct((N_SUB, H2), jnp.float32),
    mesh=_zmesh,
    scratch_types=[
        pltpu.VMEM((ZPT,), jnp.int32),
        pltpu.VMEM((ZPT, H2), jnp.float32),
        pltpu.SemaphoreType.DMA,
    ],
    compiler_params=pltpu.CompilerParams(use_tc_tiling_on_sc=False),
)
def _sc_zgather(mu_hbm, rel_hbm, out_hbm, idx_v, rows_v, sem):
    base = (lax.axis_index("c") * NS + lax.axis_index("s")) * ZPT
    pltpu.sync_copy(rel_hbm.at[pl.ds(base, ZPT)], idx_v)
    pltpu.async_copy(mu_hbm.at[idx_v], rows_v, sem).wait()
    pltpu.sync_copy(rows_v, out_hbm.at[pl.ds(base, ZPT)])


_BLK = 1024


def _mm1(xp, W1):
    def body(x_ref, w_ref, o_ref):
        o_ref[...] = jnp.dot(x_ref[...], w_ref[...],
                             preferred_element_type=jnp.float32)

    return pl.pallas_call(
        body,
        grid=(NPAD // _BLK,),
        in_specs=[pl.BlockSpec((_BLK, D_IN), lambda i: (i, 0)),
                  pl.BlockSpec((D_IN, H1), lambda i: (0, 0))],
        out_specs=pl.BlockSpec((_BLK, H1), lambda i: (i, 0)),
        out_shape=jax.ShapeDtypeStruct((NPAD, H1), jnp.float32),
    )(xp, W1)


def _merge1(p0, p1, yt, d0, d1, b1_2d):
    def body(p0_ref, p1_ref, yt_ref, d0_ref, d1_ref, b_ref, h_ref, inv_ref):
        den = d0_ref[:, 0:1] + d1_ref[:, 0:1] + 1.0
        inv = 1.0 / den
        num = p0_ref[...] + p1_ref[...] - yt_ref[...]
        h_ref[...] = jnp.maximum(num * inv + b_ref[0:1, :], 0.0)
        inv_ref[...] = inv

    return pl.pallas_call(
        body,
        grid=(NPAD // _BLK,),
        in_specs=[pl.BlockSpec((_BLK, H1), lambda i: (i, 0)),
                  pl.BlockSpec((_BLK, H1), lambda i: (i, 0)),
                  pl.BlockSpec((_BLK, H1), lambda i: (i, 0)),
                  pl.BlockSpec((_BLK, DW), lambda i: (i, 0)),
                  pl.BlockSpec((_BLK, DW), lambda i: (i, 0)),
                  pl.BlockSpec((8, H1), lambda i: (0, 0))],
        out_specs=[pl.BlockSpec((_BLK, H1), lambda i: (i, 0)),
                   pl.BlockSpec((_BLK, 1), lambda i: (i, 0))],
        out_shape=[jax.ShapeDtypeStruct((NPAD, H1), jnp.float32),
                   jax.ShapeDtypeStruct((NPAD, 1), jnp.float32)],
    )(p0, p1, yt, d0, d1, b1_2d)


def _heads(q0, q1, h1, inv, W2, b2_2d, W3, b3_2d):
    def body(q0_ref, q1_ref, h_ref, inv_ref, w2_ref, b2_ref, w3_ref, b3_ref,
             mu_ref, lv_ref):
        nbar = (q0_ref[...] + q1_ref[...] - h_ref[...]) * inv_ref[...]
        mu_ref[...] = jnp.dot(nbar, w2_ref[...],
                              preferred_element_type=jnp.float32) + b2_ref[0:1, :]
        lv_ref[...] = jnp.dot(nbar, w3_ref[...],
                              preferred_element_type=jnp.float32) + b3_ref[0:1, :]

    return pl.pallas_call(
        body,
        grid=(NPAD // _BLK,),
        in_specs=[pl.BlockSpec((_BLK, H1), lambda i: (i, 0)),
                  pl.BlockSpec((_BLK, H1), lambda i: (i, 0)),
                  pl.BlockSpec((_BLK, H1), lambda i: (i, 0)),
                  pl.BlockSpec((_BLK, 1), lambda i: (i, 0)),
                  pl.BlockSpec((H1, H2), lambda i: (0, 0)),
                  pl.BlockSpec((8, H2), lambda i: (0, 0)),
                  pl.BlockSpec((H1, H2), lambda i: (0, 0)),
                  pl.BlockSpec((8, H2), lambda i: (0, 0))],
        out_specs=[pl.BlockSpec((_BLK, H2), lambda i: (i, 0)),
                   pl.BlockSpec((_BLK, H2), lambda i: (i, 0))],
        out_shape=[jax.ShapeDtypeStruct((NPAD, H2), jnp.float32),
                   jax.ShapeDtypeStruct((NPAD, H2), jnp.float32)],
    )(q0, q1, h1, inv, W2, b2_2d, W3, b3_2d)


def _decode(z):
    def body(z_ref, o_ref):
        zz = z_ref[...]
        o_ref[...] = lax.dot_general(zz, zz, (((1,), (1,)), ((), ())),
                                     preferred_element_type=jnp.float32)

    return pl.pallas_call(
        body,
        out_shape=jax.ShapeDtypeStruct((N_SUB, N_SUB), jnp.float32),
    )(z)


def kernel(features, edge_index, relative_node_idx, W1, b1, W2, b2, W3, b3):
    src = edge_index[0]
    dst = edge_index[1]
    epad = EPAD - N_EDGES
    src_p = jnp.concatenate([src, jnp.zeros((epad,), jnp.int32)]
                            ).reshape(EPAD // ECHUNK, ECHUNK)
    pad_dst = N_NODES + jnp.arange(epad, dtype=jnp.int32) % (NPAD - N_NODES)
    dst_p = jnp.concatenate([dst, pad_dst]).reshape(EPAD // ECHUNK, ECHUNK)
    xp = jnp.pad(features, ((0, NPAD - N_NODES), (0, 0)))
    b1_2d = jnp.broadcast_to(b1, (8, H1))
    b2_2d = jnp.broadcast_to(b2, (8, H2))
    b3_2d = jnp.broadcast_to(b3, (8, H2))

    yt = _mm1(xp, W1)
    p, degp = _sc_pass1(yt, src_p, dst_p)
    h1, inv = _merge1(p[0], p[1], yt, degp[0], degp[1], b1_2d)
    q, = _sc_pass2(h1, src_p, dst_p)
    mu_full, lv_full = _heads(q[0], q[1], h1, inv, W2, b2_2d, W3, b3_2d)
    z = _sc_zgather(mu_full, relative_node_idx)
    recovered = _decode(z)
    return recovered, mu_full[:N_NODES], lv_full[:N_NODES]


# speedup vs baseline: 14.8352x; 1.0430x over previous
import functools

import jax
import jax.numpy as jnp
from jax import lax
from jax.experimental import pallas as pl
from jax.experimental.pallas import tpu as pltpu
from jax.experimental.pallas import tpu_sc as plsc

N_NODES = 10000
N_EDGES = 320000
D_IN = 128
H1 = 32
H2 = 16
N_SUB = 1024

NC = 2
NS = 16
NW = NC * NS
L = 16

NPAD = 10240
EPAD = NW * 10240
EPT = EPAD // NW
ECHUNK = 128
NCHUNK = EPT // ECHUNK
RPT = NPAD // NS
ZPT = N_SUB // NW
DW = 16
KPIPE = 8
NSUPER = NCHUNK // KPIPE

_MESH = dict(mesh=plsc.VectorSubcoreMesh(core_axis_name="c",
                                         subcore_axis_name="s"),
             compiler_params=pltpu.CompilerParams(use_tc_tiling_on_sc=False))


def _edge_loop(tab_sh, acc_sh, src_v, dst_v, rows_v, gsem, ssem,
               deg_sh=None, ones_v=None):
    def superchunk(s, carry):
        c0 = s * KPIPE
        for j in range(KPIPE):
            pltpu.async_copy(tab_sh.at[src_v.at[c0 + j]], rows_v.at[j], gsem)
        for j in range(KPIPE):
            pltpu.make_async_copy(tab_sh.at[src_v.at[c0 + j]], rows_v.at[j],
                                  gsem).wait()
        for j in range(KPIPE):
            pltpu.async_copy(rows_v.at[j], acc_sh.at[dst_v.at[c0 + j]],
                             ssem, add=True)
            if deg_sh is not None:
                pltpu.async_copy(ones_v, deg_sh.at[dst_v.at[c0 + j]],
                                 ssem, add=True)
        for j in range(KPIPE):
            pltpu.make_async_copy(rows_v.at[j], acc_sh.at[dst_v.at[c0 + j]],
                                  ssem).wait()
            if deg_sh is not None:
                pltpu.make_async_copy(ones_v, deg_sh.at[dst_v.at[c0 + j]],
                                      ssem).wait()
        return carry

    lax.fori_loop(0, NSUPER, superchunk, 0)


@functools.partial(
    pl.kernel,
    out_type=(jax.ShapeDtypeStruct((NC, NPAD, H1), jnp.float32),
              jax.ShapeDtypeStruct((NC, NPAD, DW), jnp.float32),
              jax.ShapeDtypeStruct((NC, N_SUB, DW), jnp.float32)),
    scratch_types=[
        pltpu.VMEM_SHARED((NPAD, H1), jnp.float32),
        pltpu.VMEM_SHARED((NPAD, H1), jnp.float32),
        pltpu.VMEM_SHARED((NPAD, DW), jnp.float32),
        pltpu.VMEM((NCHUNK, ECHUNK), jnp.int32),
        pltpu.VMEM((NCHUNK, ECHUNK), jnp.int32),
        pltpu.VMEM((KPIPE, ECHUNK, H1), jnp.float32),
        pltpu.VMEM((ECHUNK, DW), jnp.float32),
        pltpu.VMEM((ZPT,), jnp.int32),
        pltpu.VMEM((ZPT, DW), jnp.float32),
        pltpu.SemaphoreType.DMA,
        pltpu.SemaphoreType.DMA,
    ],
    **_MESH)
def _sc_pass1(table_hbm, src_hbm, dst_hbm, rel_hbm,
              out_hbm, degp_hbm, zdeg_hbm,
              acc_sh, tab_sh, deg_sh, src_v, dst_v, rows_v, ones_v,
              zidx_v, zrows_v, gsem, ssem):
    cid = lax.axis_index("c")
    sid = lax.axis_index("s")
    rbase = sid * RPT
    tbase = (cid * NS + sid) * NCHUNK
    pltpu.sync_copy(src_hbm.at[pl.ds(tbase, NCHUNK)], src_v)
    pltpu.sync_copy(dst_hbm.at[pl.ds(tbase, NCHUNK)], dst_v)
    pltpu.sync_copy(table_hbm.at[pl.ds(rbase, RPT)],
                    acc_sh.at[pl.ds(rbase, RPT)])
    pltpu.sync_copy(table_hbm.at[pl.ds(rbase, RPT)],
                    tab_sh.at[pl.ds(rbase, RPT)])
    zeros16 = jnp.zeros((L,), jnp.float32)
    ones16 = jnp.full((L,), 1.0, jnp.float32)

    def zloop(i, c):
        ones_v[i, :] = zeros16
        return c

    lax.fori_loop(0, ECHUNK, zloop, 0)
    for k in range(RPT // ECHUNK):
        pltpu.sync_copy(ones_v, deg_sh.at[pl.ds(rbase + k * ECHUNK, ECHUNK)])

    def floop(i, c):
        ones_v[i, :] = ones16
        return c

    lax.fori_loop(0, ECHUNK, floop, 0)
    plsc.subcore_barrier()

    _edge_loop(tab_sh, acc_sh, src_v, dst_v, rows_v, gsem, ssem,
               deg_sh=deg_sh, ones_v=ones_v)
    plsc.subcore_barrier()

    pltpu.sync_copy(acc_sh.at[pl.ds(rbase, RPT)],
                    out_hbm.at[cid].at[pl.ds(rbase, RPT)])
    pltpu.sync_copy(deg_sh.at[pl.ds(rbase, RPT)],
                    degp_hbm.at[cid].at[pl.ds(rbase, RPT)])
    zbase = (cid * NS + sid) * ZPT
    pltpu.sync_copy(rel_hbm.at[pl.ds(zbase, ZPT)], zidx_v)
    pltpu.async_copy(deg_sh.at[zidx_v], zrows_v, gsem).wait()
    pltpu.sync_copy(zrows_v, zdeg_hbm.at[cid].at[pl.ds(zbase, ZPT)])


@functools.partial(
    pl.kernel,
    out_type=(jax.ShapeDtypeStruct((NC, NPAD, H1), jnp.float32),
              jax.ShapeDtypeStruct((NPAD, H1), jnp.float32),
              jax.ShapeDtypeStruct((NC, N_SUB, H1), jnp.float32),
              jax.ShapeDtypeStruct((N_SUB, H1), jnp.float32)),
    scratch_types=[
        pltpu.VMEM_SHARED((NPAD, H1), jnp.float32),
        pltpu.VMEM_SHARED((NPAD, H1), jnp.float32),
        pltpu.VMEM((NCHUNK, ECHUNK), jnp.int32),
        pltpu.VMEM((NCHUNK, ECHUNK), jnp.int32),
        pltpu.VMEM((KPIPE, ECHUNK, H1), jnp.float32),
        pltpu.VMEM((ECHUNK, H1), jnp.float32),
        pltpu.VMEM((ECHUNK, H1), jnp.float32),
        pltpu.VMEM((ECHUNK, H1), jnp.float32),
        pltpu.VMEM((ECHUNK, DW), jnp.float32),
        pltpu.VMEM((ECHUNK, DW), jnp.float32),
        pltpu.VMEM((H1,), jnp.float32),
        pltpu.VMEM((ZPT,), jnp.int32),
        pltpu.VMEM((ZPT, H1), jnp.float32),
        pltpu.SemaphoreType.DMA,
        pltpu.SemaphoreType.DMA,
    ],
    **_MESH)
def _sc_pass2(p_hbm, degp_hbm, y_hbm, b1_hbm, src_hbm, dst_hbm, rel_hbm,
              out_hbm, h1_hbm, zq_hbm, zh_hbm,
              acc_sh, tab_sh, src_v, dst_v, rows_v,
              pa_v, pb_v, yc_v, da_v, db_v, b1_v,
              zidx_v, zrows_v, gsem, ssem):
    cid = lax.axis_index("c")
    sid = lax.axis_index("s")
    rbase = sid * RPT
    tbase = (cid * NS + sid) * NCHUNK
    pltpu.sync_copy(src_hbm.at[pl.ds(tbase, NCHUNK)], src_v)
    pltpu.sync_copy(dst_hbm.at[pl.ds(tbase, NCHUNK)], dst_v)
    pltpu.sync_copy(b1_hbm, b1_v)

    for k in range(RPT // ECHUNK):
        r0 = rbase + k * ECHUNK
        pltpu.sync_copy(p_hbm.at[0].at[pl.ds(r0, ECHUNK)], pa_v)
        pltpu.sync_copy(p_hbm.at[1].at[pl.ds(r0, ECHUNK)], pb_v)
        pltpu.sync_copy(y_hbm.at[pl.ds(r0, ECHUNK)], yc_v)
        pltpu.sync_copy(degp_hbm.at[0].at[pl.ds(r0, ECHUNK)], da_v)
        pltpu.sync_copy(degp_hbm.at[1].at[pl.ds(r0, ECHUNK)], db_v)

        def rowloop(r, c):
            inv16 = 1.0 / (da_v[r, :] + db_v[r, :] + 1.0)
            for half in range(H1 // L):
                sl = pl.ds(half * L, L)
                h16 = jnp.maximum(
                    (pa_v[r, sl] + pb_v[r, sl] - yc_v[r, sl]) * inv16
                    + b1_v[sl], 0.0)
                pa_v[r, sl] = h16
            return c

        lax.fori_loop(0, ECHUNK, rowloop, 0)
        pltpu.sync_copy(pa_v, tab_sh.at[pl.ds(r0, ECHUNK)])
        pltpu.sync_copy(pa_v, acc_sh.at[pl.ds(r0, ECHUNK)])

        @pl.when(cid == 0)
        def _():
            pltpu.sync_copy(pa_v, h1_hbm.at[pl.ds(r0, ECHUNK)])

    plsc.subcore_barrier()

    _edge_loop(tab_sh, acc_sh, src_v, dst_v, rows_v, gsem, ssem)
    plsc.subcore_barrier()

    pltpu.sync_copy(acc_sh.at[pl.ds(rbase, RPT)],
                    out_hbm.at[cid].at[pl.ds(rbase, RPT)])
    zbase = (cid * NS + sid) * ZPT
    pltpu.sync_copy(rel_hbm.at[pl.ds(zbase, ZPT)], zidx_v)
    pltpu.async_copy(acc_sh.at[zidx_v], zrows_v, gsem).wait()
    pltpu.sync_copy(zrows_v, zq_hbm.at[cid].at[pl.ds(zbase, ZPT)])
    pltpu.async_copy(tab_sh.at[zidx_v], zrows_v, gsem).wait()
    pltpu.sync_copy(zrows_v, zh_hbm.at[pl.ds(zbase, ZPT)])


_BLK = 1024


def _mm1(xp, W1):
    def body(x_ref, w_ref, o_ref):
        o_ref[...] = jnp.dot(x_ref[...], w_ref[...],
                             preferred_element_type=jnp.float32)

    return pl.pallas_call(
        body,
        grid=(NPAD // _BLK,),
        in_specs=[pl.BlockSpec((_BLK, D_IN), lambda i: (i, 0)),
                  pl.BlockSpec((D_IN, H1), lambda i: (0, 0))],
        out_specs=pl.BlockSpec((_BLK, H1), lambda i: (i, 0)),
        out_shape=jax.ShapeDtypeStruct((NPAD, H1), jnp.float32),
    )(xp, W1)


def _heads(q0, q1, h1, d0, d1, W2, b2_2d, W3, b3_2d):
    def body(q0_ref, q1_ref, h_ref, d0_ref, d1_ref, w2_ref, b2_ref, w3_ref,
             b3_ref, mu_ref, lv_ref):
        inv = 1.0 / (d0_ref[:, 0:1] + d1_ref[:, 0:1] + 1.0)
        nbar = (q0_ref[...] + q1_ref[...] - h_ref[...]) * inv
        mu_ref[...] = jnp.dot(nbar, w2_ref[...],
                              preferred_element_type=jnp.float32) + b2_ref[0:1, :]
        lv_ref[...] = jnp.dot(nbar, w3_ref[...],
                              preferred_element_type=jnp.float32) + b3_ref[0:1, :]

    return pl.pallas_call(
        body,
        grid=(NPAD // _BLK,),
        in_specs=[pl.BlockSpec((_BLK, H1), lambda i: (i, 0)),
                  pl.BlockSpec((_BLK, H1), lambda i: (i, 0)),
                  pl.BlockSpec((_BLK, H1), lambda i: (i, 0)),
                  pl.BlockSpec((_BLK, DW), lambda i: (i, 0)),
                  pl.BlockSpec((_BLK, DW), lambda i: (i, 0)),
                  pl.BlockSpec((H1, H2), lambda i: (0, 0)),
                  pl.BlockSpec((8, H2), lambda i: (0, 0)),
                  pl.BlockSpec((H1, H2), lambda i: (0, 0)),
                  pl.BlockSpec((8, H2), lambda i: (0, 0))],
        out_specs=[pl.BlockSpec((_BLK, H2), lambda i: (i, 0)),
                   pl.BlockSpec((_BLK, H2), lambda i: (i, 0))],
        out_shape=[jax.ShapeDtypeStruct((NPAD, H2), jnp.float32),
                   jax.ShapeDtypeStruct((NPAD, H2), jnp.float32)],
    )(q0, q1, h1, d0, d1, W2, b2_2d, W3, b3_2d)


def _decode(zq0, zq1, zh, zd0, zd1, W2, b2_2d):
    def body(zq0_ref, zq1_ref, zh_ref, zd0_ref, zd1_ref, w2_ref, b2_ref,
             o_ref):
        zinv = 1.0 / (zd0_ref[:, 0:1] + zd1_ref[:, 0:1] + 1.0)
        znbar = (zq0_ref[...] + zq1_ref[...] - zh_ref[...]) * zinv
        z = jnp.dot(znbar, w2_ref[...],
                    preferred_element_type=jnp.float32) + b2_ref[0:1, :]
        o_ref[...] = lax.dot_general(z, z, (((1,), (1,)), ((), ())),
                                     preferred_element_type=jnp.float32)

    return pl.pallas_call(
        body,
        out_shape=jax.ShapeDtypeStruct((N_SUB, N_SUB), jnp.float32),
    )(zq0, zq1, zh, zd0, zd1, W2, b2_2d)


def kernel(features, edge_index, relative_node_idx, W1, b1, W2, b2, W3, b3):
    src = edge_index[0]
    dst = edge_index[1]
    epad = EPAD - N_EDGES
    src_p = jnp.concatenate([src, jnp.zeros((epad,), jnp.int32)]
                            ).reshape(EPAD // ECHUNK, ECHUNK)
    pad_dst = N_NODES + jnp.arange(epad, dtype=jnp.int32) % (NPAD - N_NODES)
    dst_p = jnp.concatenate([dst, pad_dst]).reshape(EPAD // ECHUNK, ECHUNK)
    xp = jnp.pad(features, ((0, NPAD - N_NODES), (0, 0)))
    b2_2d = jnp.broadcast_to(b2, (8, H2))
    b3_2d = jnp.broadcast_to(b3, (8, H2))

    yt = _mm1(xp, W1)
    p, degp, zdeg = _sc_pass1(yt, src_p, dst_p, relative_node_idx)
    q, h1, zq, zh = _sc_pass2(p, degp, yt, b1, src_p, dst_p,
                              relative_node_idx)
    mu_full, lv_full = _heads(q[0], q[1], h1, degp[0], degp[1],
                              W2, b2_2d, W3, b3_2d)
    recovered = _decode(zq[0], zq[1], zh, zdeg[0], zdeg[1], W2, b2_2d)
    return recovered, mu_full[:N_NODES], lv_full[:N_NODES]
